# baseline (speedup 1.0000x reference)
import jax
import jax.numpy as jnp
from jax import lax
from jax.experimental import pallas as pl
from jax.experimental.pallas import tpu as pltpu

M, P, C_IN, C_OUT = 40000, 32, 4, 64
LANES = P * C_IN
WIDE = P * C_OUT
BM = 1000
NB = M // BM
EPS = 1e-5


def _vfe_kernel(x_ref, n_ref, w_ref, wbig_ref, sel_ref, selt_ref,
                e_ref, bd_ref, b_ref, gamma_ref, beta_ref, out_ref,
                g_acc, s_acc, cnt_acc, w2_ref, t_ref, rt_ref):
    ph = pl.program_id(0)
    i = pl.program_id(1)

    n = n_ref[:]
    count = jnp.clip(n, 0.0, float(P))
    pidx = (lax.broadcasted_iota(jnp.int32, (BM, LANES), 1) // C_IN
            ).astype(jnp.float32)
    xm = jnp.where(pidx < n, x_ref[:], 0.0)

    @pl.when(jnp.logical_and(ph == 0, i == 0))
    def _init():
        g_acc[:] = jnp.zeros_like(g_acc)
        s_acc[:] = jnp.zeros_like(s_acc)
        cnt_acc[0, 0] = 0.0

    @pl.when(ph == 0)
    def _accumulate():
        g_acc[:] += lax.dot_general(
            xm, xm, (((0,), (0,)), ((), ())),
            preferred_element_type=jnp.float32)
        s_acc[:] += jnp.sum(xm, axis=0, keepdims=True)
        cnt_acc[0, 0] += jnp.sum(count)

    @pl.when(jnp.logical_and(ph == 1, i == 0))
    def _finalize_stats():
        cnt = cnt_acc[0, 0]
        nv = jnp.maximum(cnt, 1.0)
        gm = g_acc[:] * bd_ref[:]
        sxx = jnp.dot(
            lax.dot_general(e_ref[:], gm, (((0,), (0,)), ((), ())),
                            preferred_element_type=jnp.float32),
            e_ref[:], preferred_element_type=jnp.float32)
        s4 = jnp.dot(s_acc[:], e_ref[:],
                     preferred_element_type=jnp.float32)
        sw = jnp.dot(s4, w_ref[:],
                     preferred_element_type=jnp.float32)
        bvec = b_ref[:]
        mean = (sw + cnt * bvec) / nv
        t4 = jnp.dot(sxx, w_ref[:],
                     preferred_element_type=jnp.float32)
        q = (jnp.sum(w_ref[:] * t4, axis=0, keepdims=True)
             + 2.0 * bvec * sw + cnt * bvec * bvec)
        var = q / nv - mean * mean
        scale = gamma_ref[:] * lax.rsqrt(var + EPS)
        shift = beta_ref[:] - mean * scale
        tb = scale * bvec + shift
        rt_ref[:] = jnp.maximum(tb, 0.0)
        s2048 = jnp.dot(scale, selt_ref[:],
                        preferred_element_type=jnp.float32)
        t_ref[:] = jnp.dot(tb, selt_ref[:],
                           preferred_element_type=jnp.float32)
        w2_ref[:] = (wbig_ref[:] * s2048).astype(jnp.bfloat16)

    @pl.when(ph == 1)
    def _emit():
        act = jnp.maximum(
            jnp.dot(xm.astype(jnp.bfloat16), w2_ref[:],
                    preferred_element_type=jnp.float32)
            + t_ref[:], 0.0)
        summed = jnp.dot(act.astype(jnp.bfloat16), sel_ref[:],
                         preferred_element_type=jnp.float32)
        summed = summed - (float(P) - count) * rt_ref[:]
        inv = jnp.where(count > 0.0, 1.0 / jnp.maximum(count, 1.0), 0.0)
        out_ref[:] = summed * inv


@jax.jit
def _vfe(x2d, nf, w, wbig, sel, selt, e, bd, b2, gamma2, beta2):
    return pl.pallas_call(
        _vfe_kernel,
        grid=(2, NB),
        in_specs=[
            pl.BlockSpec((BM, LANES), lambda ph, i: (i, 0)),
            pl.BlockSpec((BM, 1), lambda ph, i: (i, 0)),
            pl.BlockSpec((C_IN, C_OUT), lambda ph, i: (0, 0)),
            pl.BlockSpec((LANES, WIDE), lambda ph, i: (0, 0)),
            pl.BlockSpec((WIDE, C_OUT), lambda ph, i: (0, 0)),
            pl.BlockSpec((C_OUT, WIDE), lambda ph, i: (0, 0)),
            pl.BlockSpec((LANES, C_IN), lambda ph, i: (0, 0)),
            pl.BlockSpec((LANES, LANES), lambda ph, i: (0, 0)),
            pl.BlockSpec((1, C_OUT), lambda ph, i: (0, 0)),
            pl.BlockSpec((1, C_OUT), lambda ph, i: (0, 0)),
            pl.BlockSpec((1, C_OUT), lambda ph, i: (0, 0)),
        ],
        out_specs=pl.BlockSpec((BM, C_OUT), lambda ph, i: (ph * i, 0)),
        out_shape=jax.ShapeDtypeStruct((M, C_OUT), jnp.float32),
        scratch_shapes=[
            pltpu.VMEM((LANES, LANES), jnp.float32),
            pltpu.VMEM((1, LANES), jnp.float32),
            pltpu.SMEM((1, 1), jnp.float32),
            pltpu.VMEM((LANES, WIDE), jnp.bfloat16),
            pltpu.VMEM((1, WIDE), jnp.float32),
            pltpu.VMEM((1, C_OUT), jnp.float32),
        ],
    )(x2d, nf, w, wbig, sel, selt, e, bd, b2, gamma2, beta2)


def kernel(voxel_features, voxel_num_points, W, b, gamma, beta):
    x2d = voxel_features.reshape(M, LANES)
    nf = jnp.asarray(voxel_num_points).astype(jnp.float32).reshape(M, 1)
    eye_p = jnp.eye(P, dtype=jnp.float32)
    wbig = jnp.kron(eye_p, W)
    self32 = jnp.kron(jnp.ones((P, 1), jnp.float32),
                      jnp.eye(C_OUT, dtype=jnp.float32))
    sel = self32.astype(jnp.bfloat16)
    selt = self32.T
    e = jnp.tile(jnp.eye(C_IN, dtype=jnp.float32), (P, 1))
    bd = jnp.kron(eye_p, jnp.ones((C_IN, C_IN), jnp.float32))
    return _vfe(x2d, nf, W, wbig, sel, selt, e, bd,
                b.reshape(1, C_OUT), gamma.reshape(1, C_OUT),
                beta.reshape(1, C_OUT))


# speedup vs baseline: 1.0965x; 1.0965x over previous
import jax
import jax.numpy as jnp
from jax import lax
from jax.experimental import pallas as pl
from jax.experimental.pallas import tpu as pltpu

M, P, C_IN, C_OUT = 40000, 32, 4, 64
LANES = P * C_IN
WIDE = P * C_OUT
BM = 2000
NB = M // BM
EPS = 1e-5


def _vfe_kernel(x_ref, n_ref, w_ref, wbig_ref, sel_ref, selt_ref,
                e_ref, bd_ref, b_ref, gamma_ref, beta_ref, out_ref,
                g_acc, s_acc, cnt_acc, w2_ref, t_ref, rt_ref):
    ph = pl.program_id(0)
    i = pl.program_id(1)

    n = n_ref[:]
    count = jnp.clip(n, 0.0, float(P))
    pidx = (lax.broadcasted_iota(jnp.int32, (BM, LANES), 1) // C_IN
            ).astype(jnp.float32)
    xm = jnp.where(pidx < n, x_ref[:], 0.0)

    @pl.when(jnp.logical_and(ph == 0, i == 0))
    def _init():
        g_acc[:] = jnp.zeros_like(g_acc)
        s_acc[:] = jnp.zeros_like(s_acc)
        cnt_acc[0, 0] = 0.0

    @pl.when(ph == 0)
    def _accumulate():
        g_acc[:] += lax.dot_general(
            xm, xm, (((0,), (0,)), ((), ())),
            preferred_element_type=jnp.float32)
        s_acc[:] += jnp.sum(xm, axis=0, keepdims=True)
        cnt_acc[0, 0] += jnp.sum(count)

    @pl.when(jnp.logical_and(ph == 1, i == 0))
    def _finalize_stats():
        cnt = cnt_acc[0, 0]
        nv = jnp.maximum(cnt, 1.0)
        gm = g_acc[:] * bd_ref[:]
        sxx = jnp.dot(
            lax.dot_general(e_ref[:], gm, (((0,), (0,)), ((), ())),
                            preferred_element_type=jnp.float32),
            e_ref[:], preferred_element_type=jnp.float32)
        s4 = jnp.dot(s_acc[:], e_ref[:],
                     preferred_element_type=jnp.float32)
        sw = jnp.dot(s4, w_ref[:],
                     preferred_element_type=jnp.float32)
        bvec = b_ref[:]
        mean = (sw + cnt * bvec) / nv
        t4 = jnp.dot(sxx, w_ref[:],
                     preferred_element_type=jnp.float32)
        q = (jnp.sum(w_ref[:] * t4, axis=0, keepdims=True)
             + 2.0 * bvec * sw + cnt * bvec * bvec)
        var = q / nv - mean * mean
        scale = gamma_ref[:] * lax.rsqrt(var + EPS)
        shift = beta_ref[:] - mean * scale
        tb = scale * bvec + shift
        rt_ref[:] = jnp.maximum(tb, 0.0)
        s2048 = jnp.dot(scale, selt_ref[:],
                        preferred_element_type=jnp.float32)
        t_ref[:] = jnp.dot(tb, selt_ref[:],
                           preferred_element_type=jnp.float32)
        w2_ref[:] = (wbig_ref[:] * s2048).astype(jnp.bfloat16)

    @pl.when(ph == 1)
    def _emit():
        act = jnp.maximum(
            jnp.dot(xm.astype(jnp.bfloat16), w2_ref[:],
                    preferred_element_type=jnp.float32)
            + t_ref[:], 0.0)
        summed = jnp.dot(act.astype(jnp.bfloat16), sel_ref[:],
                         preferred_element_type=jnp.float32)
        summed = summed - (float(P) - count) * rt_ref[:]
        inv = jnp.where(count > 0.0, 1.0 / jnp.maximum(count, 1.0), 0.0)
        out_ref[:] = summed * inv


@jax.jit
def _vfe(x2d, nf, w, wbig, sel, selt, e, bd, b2, gamma2, beta2):
    return pl.pallas_call(
        _vfe_kernel,
        grid=(2, NB),
        in_specs=[
            pl.BlockSpec((BM, LANES), lambda ph, i: (i, 0)),
            pl.BlockSpec((BM, 1), lambda ph, i: (i, 0)),
            pl.BlockSpec((C_IN, C_OUT), lambda ph, i: (0, 0)),
            pl.BlockSpec((LANES, WIDE), lambda ph, i: (0, 0)),
            pl.BlockSpec((WIDE, C_OUT), lambda ph, i: (0, 0)),
            pl.BlockSpec((C_OUT, WIDE), lambda ph, i: (0, 0)),
            pl.BlockSpec((LANES, C_IN), lambda ph, i: (0, 0)),
            pl.BlockSpec((LANES, LANES), lambda ph, i: (0, 0)),
            pl.BlockSpec((1, C_OUT), lambda ph, i: (0, 0)),
            pl.BlockSpec((1, C_OUT), lambda ph, i: (0, 0)),
            pl.BlockSpec((1, C_OUT), lambda ph, i: (0, 0)),
        ],
        out_specs=pl.BlockSpec((BM, C_OUT), lambda ph, i: (ph * i, 0)),
        out_shape=jax.ShapeDtypeStruct((M, C_OUT), jnp.float32),
        scratch_shapes=[
            pltpu.VMEM((LANES, LANES), jnp.float32),
            pltpu.VMEM((1, LANES), jnp.float32),
            pltpu.SMEM((1, 1), jnp.float32),
            pltpu.VMEM((LANES, WIDE), jnp.bfloat16),
            pltpu.VMEM((1, WIDE), jnp.float32),
            pltpu.VMEM((1, C_OUT), jnp.float32),
        ],
    )(x2d, nf, w, wbig, sel, selt, e, bd, b2, gamma2, beta2)


def kernel(voxel_features, voxel_num_points, W, b, gamma, beta):
    x2d = voxel_features.reshape(M, LANES)
    nf = jnp.asarray(voxel_num_points).astype(jnp.float32).reshape(M, 1)
    eye_p = jnp.eye(P, dtype=jnp.float32)
    wbig = jnp.kron(eye_p, W)
    self32 = jnp.kron(jnp.ones((P, 1), jnp.float32),
                      jnp.eye(C_OUT, dtype=jnp.float32))
    sel = self32.astype(jnp.bfloat16)
    selt = self32.T
    e = jnp.tile(jnp.eye(C_IN, dtype=jnp.float32), (P, 1))
    bd = jnp.kron(eye_p, jnp.ones((C_IN, C_IN), jnp.float32))
    return _vfe(x2d, nf, W, wbig, sel, selt, e, bd,
                b.reshape(1, C_OUT), gamma.reshape(1, C_OUT),
                beta.reshape(1, C_OUT))


# speedup vs baseline: 1.1465x; 1.0456x over previous
import jax
import jax.numpy as jnp
from jax import lax
from jax.experimental import pallas as pl
from jax.experimental.pallas import tpu as pltpu

M, P, C_IN, C_OUT = 40000, 32, 4, 64
LANES = P * C_IN
WIDE = P * C_OUT
BM = 4000
NB = M // BM
EPS = 1e-5


def _vfe_kernel(x_ref, n_ref, w_ref, wbig_ref, sel_ref, selt_ref,
                e_ref, bd_ref, b_ref, gamma_ref, beta_ref, out_ref,
                g_acc, s_acc, cnt_acc, w2_ref, t_ref, rt_ref):
    ph = pl.program_id(0)
    i = pl.program_id(1)

    n = n_ref[:]
    count = jnp.clip(n, 0.0, float(P))
    pidx = (lax.broadcasted_iota(jnp.int32, (BM, LANES), 1) // C_IN
            ).astype(jnp.float32)
    xm = jnp.where(pidx < n, x_ref[:], 0.0)

    @pl.when(jnp.logical_and(ph == 0, i == 0))
    def _init():
        g_acc[:] = jnp.zeros_like(g_acc)
        s_acc[:] = jnp.zeros_like(s_acc)
        cnt_acc[0, 0] = 0.0

    @pl.when(ph == 0)
    def _accumulate():
        g_acc[:] += lax.dot_general(
            xm, xm, (((0,), (0,)), ((), ())),
            preferred_element_type=jnp.float32)
        s_acc[:] += jnp.sum(xm, axis=0, keepdims=True)
        cnt_acc[0, 0] += jnp.sum(count)

    @pl.when(jnp.logical_and(ph == 1, i == 0))
    def _finalize_stats():
        cnt = cnt_acc[0, 0]
        nv = jnp.maximum(cnt, 1.0)
        gm = g_acc[:] * bd_ref[:]
        sxx = jnp.dot(
            lax.dot_general(e_ref[:], gm, (((0,), (0,)), ((), ())),
                            preferred_element_type=jnp.float32),
            e_ref[:], preferred_element_type=jnp.float32)
        s4 = jnp.dot(s_acc[:], e_ref[:],
                     preferred_element_type=jnp.float32)
        sw = jnp.dot(s4, w_ref[:],
                     preferred_element_type=jnp.float32)
        bvec = b_ref[:]
        mean = (sw + cnt * bvec) / nv
        t4 = jnp.dot(sxx, w_ref[:],
                     preferred_element_type=jnp.float32)
        q = (jnp.sum(w_ref[:] * t4, axis=0, keepdims=True)
             + 2.0 * bvec * sw + cnt * bvec * bvec)
        var = q / nv - mean * mean
        scale = gamma_ref[:] * lax.rsqrt(var + EPS)
        shift = beta_ref[:] - mean * scale
        tb = scale * bvec + shift
        rt_ref[:] = jnp.maximum(tb, 0.0)
        s2048 = jnp.dot(scale, selt_ref[:],
                        preferred_element_type=jnp.float32)
        t_ref[:] = jnp.dot(tb, selt_ref[:],
                           preferred_element_type=jnp.float32)
        w2_ref[:] = (wbig_ref[:] * s2048).astype(jnp.bfloat16)

    @pl.when(ph == 1)
    def _emit():
        act = jnp.maximum(
            jnp.dot(xm.astype(jnp.bfloat16), w2_ref[:],
                    preferred_element_type=jnp.float32)
            + t_ref[:], 0.0)
        summed = jnp.dot(act.astype(jnp.bfloat16), sel_ref[:],
                         preferred_element_type=jnp.float32)
        summed = summed - (float(P) - count) * rt_ref[:]
        inv = jnp.where(count > 0.0, 1.0 / jnp.maximum(count, 1.0), 0.0)
        out_ref[:] = summed * inv


@jax.jit
def _vfe(x2d, nf, w, wbig, sel, selt, e, bd, b2, gamma2, beta2):
    return pl.pallas_call(
        _vfe_kernel,
        grid=(2, NB),
        in_specs=[
            pl.BlockSpec((BM, LANES), lambda ph, i: (i, 0)),
            pl.BlockSpec((BM, 1), lambda ph, i: (i, 0)),
            pl.BlockSpec((C_IN, C_OUT), lambda ph, i: (0, 0)),
            pl.BlockSpec((LANES, WIDE), lambda ph, i: (0, 0)),
            pl.BlockSpec((WIDE, C_OUT), lambda ph, i: (0, 0)),
            pl.BlockSpec((C_OUT, WIDE), lambda ph, i: (0, 0)),
            pl.BlockSpec((LANES, C_IN), lambda ph, i: (0, 0)),
            pl.BlockSpec((LANES, LANES), lambda ph, i: (0, 0)),
            pl.BlockSpec((1, C_OUT), lambda ph, i: (0, 0)),
            pl.BlockSpec((1, C_OUT), lambda ph, i: (0, 0)),
            pl.BlockSpec((1, C_OUT), lambda ph, i: (0, 0)),
        ],
        out_specs=pl.BlockSpec((BM, C_OUT), lambda ph, i: (ph * i, 0)),
        out_shape=jax.ShapeDtypeStruct((M, C_OUT), jnp.float32),
        scratch_shapes=[
            pltpu.VMEM((LANES, LANES), jnp.float32),
            pltpu.VMEM((1, LANES), jnp.float32),
            pltpu.SMEM((1, 1), jnp.float32),
            pltpu.VMEM((LANES, WIDE), jnp.bfloat16),
            pltpu.VMEM((1, WIDE), jnp.float32),
            pltpu.VMEM((1, C_OUT), jnp.float32),
        ],
    )(x2d, nf, w, wbig, sel, selt, e, bd, b2, gamma2, beta2)


def kernel(voxel_features, voxel_num_points, W, b, gamma, beta):
    x2d = voxel_features.reshape(M, LANES)
    nf = jnp.asarray(voxel_num_points).astype(jnp.float32).reshape(M, 1)
    eye_p = jnp.eye(P, dtype=jnp.float32)
    wbig = jnp.kron(eye_p, W)
    self32 = jnp.kron(jnp.ones((P, 1), jnp.float32),
                      jnp.eye(C_OUT, dtype=jnp.float32))
    sel = self32.astype(jnp.bfloat16)
    selt = self32.T
    e = jnp.tile(jnp.eye(C_IN, dtype=jnp.float32), (P, 1))
    bd = jnp.kron(eye_p, jnp.ones((C_IN, C_IN), jnp.float32))
    return _vfe(x2d, nf, W, wbig, sel, selt, e, bd,
                b.reshape(1, C_OUT), gamma.reshape(1, C_OUT),
                beta.reshape(1, C_OUT))


# speedup vs baseline: 1.1526x; 1.0053x over previous
import jax
import jax.numpy as jnp
from jax import lax
from jax.experimental import pallas as pl
from jax.experimental.pallas import tpu as pltpu

M, P, C_IN, C_OUT = 40000, 32, 4, 64
LANES = P * C_IN
WIDE = P * C_OUT
BM = 5000
NB = M // BM
EPS = 1e-5


def _vfe_kernel(x_ref, n_ref, w_ref, wbig_ref, sel_ref, selt_ref,
                e_ref, bd_ref, b_ref, gamma_ref, beta_ref, out_ref,
                g_acc, s_acc, cnt_acc, w2_ref, t_ref, rt_ref):
    ph = pl.program_id(0)
    i = pl.program_id(1)

    n = n_ref[:]
    count = jnp.clip(n, 0.0, float(P))
    pidx = (lax.broadcasted_iota(jnp.int32, (BM, LANES), 1) // C_IN
            ).astype(jnp.float32)
    xm = jnp.where(pidx < n, x_ref[:], 0.0)

    @pl.when(jnp.logical_and(ph == 0, i == 0))
    def _init():
        g_acc[:] = jnp.zeros_like(g_acc)
        s_acc[:] = jnp.zeros_like(s_acc)
        cnt_acc[0, 0] = 0.0

    @pl.when(ph == 0)
    def _accumulate():
        g_acc[:] += lax.dot_general(
            xm, xm, (((0,), (0,)), ((), ())),
            preferred_element_type=jnp.float32)
        s_acc[:] += jnp.sum(xm, axis=0, keepdims=True)
        cnt_acc[0, 0] += jnp.sum(count)

    @pl.when(jnp.logical_and(ph == 1, i == 0))
    def _finalize_stats():
        cnt = cnt_acc[0, 0]
        nv = jnp.maximum(cnt, 1.0)
        gm = g_acc[:] * bd_ref[:]
        sxx = jnp.dot(
            lax.dot_general(e_ref[:], gm, (((0,), (0,)), ((), ())),
                            preferred_element_type=jnp.float32),
            e_ref[:], preferred_element_type=jnp.float32)
        s4 = jnp.dot(s_acc[:], e_ref[:],
                     preferred_element_type=jnp.float32)
        sw = jnp.dot(s4, w_ref[:],
                     preferred_element_type=jnp.float32)
        bvec = b_ref[:]
        mean = (sw + cnt * bvec) / nv
        t4 = jnp.dot(sxx, w_ref[:],
                     preferred_element_type=jnp.float32)
        q = (jnp.sum(w_ref[:] * t4, axis=0, keepdims=True)
             + 2.0 * bvec * sw + cnt * bvec * bvec)
        var = q / nv - mean * mean
        scale = gamma_ref[:] * lax.rsqrt(var + EPS)
        shift = beta_ref[:] - mean * scale
        tb = scale * bvec + shift
        rt_ref[:] = jnp.maximum(tb, 0.0)
        s2048 = jnp.dot(scale, selt_ref[:],
                        preferred_element_type=jnp.float32)
        t_ref[:] = jnp.dot(tb, selt_ref[:],
                           preferred_element_type=jnp.float32)
        w2_ref[:] = (wbig_ref[:] * s2048).astype(jnp.bfloat16)

    @pl.when(ph == 1)
    def _emit():
        act = jnp.maximum(
            jnp.dot(xm.astype(jnp.bfloat16), w2_ref[:],
                    preferred_element_type=jnp.float32)
            + t_ref[:], 0.0)
        summed = jnp.dot(act.astype(jnp.bfloat16), sel_ref[:],
                         preferred_element_type=jnp.float32)
        summed = summed - (float(P) - count) * rt_ref[:]
        inv = jnp.where(count > 0.0, 1.0 / jnp.maximum(count, 1.0), 0.0)
        out_ref[:] = summed * inv


@jax.jit
def _vfe(x2d, nf, w, wbig, sel, selt, e, bd, b2, gamma2, beta2):
    return pl.pallas_call(
        _vfe_kernel,
        grid=(2, NB),
        in_specs=[
            pl.BlockSpec((BM, LANES), lambda ph, i: (i, 0)),
            pl.BlockSpec((BM, 1), lambda ph, i: (i, 0)),
            pl.BlockSpec((C_IN, C_OUT), lambda ph, i: (0, 0)),
            pl.BlockSpec((LANES, WIDE), lambda ph, i: (0, 0)),
            pl.BlockSpec((WIDE, C_OUT), lambda ph, i: (0, 0)),
            pl.BlockSpec((C_OUT, WIDE), lambda ph, i: (0, 0)),
            pl.BlockSpec((LANES, C_IN), lambda ph, i: (0, 0)),
            pl.BlockSpec((LANES, LANES), lambda ph, i: (0, 0)),
            pl.BlockSpec((1, C_OUT), lambda ph, i: (0, 0)),
            pl.BlockSpec((1, C_OUT), lambda ph, i: (0, 0)),
            pl.BlockSpec((1, C_OUT), lambda ph, i: (0, 0)),
        ],
        out_specs=pl.BlockSpec((BM, C_OUT), lambda ph, i: (ph * i, 0)),
        out_shape=jax.ShapeDtypeStruct((M, C_OUT), jnp.float32),
        scratch_shapes=[
            pltpu.VMEM((LANES, LANES), jnp.float32),
            pltpu.VMEM((1, LANES), jnp.float32),
            pltpu.SMEM((1, 1), jnp.float32),
            pltpu.VMEM((LANES, WIDE), jnp.bfloat16),
            pltpu.VMEM((1, WIDE), jnp.float32),
            pltpu.VMEM((1, C_OUT), jnp.float32),
        ],
    )(x2d, nf, w, wbig, sel, selt, e, bd, b2, gamma2, beta2)


def kernel(voxel_features, voxel_num_points, W, b, gamma, beta):
    x2d = voxel_features.reshape(M, LANES)
    nf = jnp.asarray(voxel_num_points).astype(jnp.float32).reshape(M, 1)
    eye_p = jnp.eye(P, dtype=jnp.float32)
    wbig = jnp.kron(eye_p, W)
    self32 = jnp.kron(jnp.ones((P, 1), jnp.float32),
                      jnp.eye(C_OUT, dtype=jnp.float32))
    sel = self32.astype(jnp.bfloat16)
    selt = self32.T
    e = jnp.tile(jnp.eye(C_IN, dtype=jnp.float32), (P, 1))
    bd = jnp.kron(eye_p, jnp.ones((C_IN, C_IN), jnp.float32))
    return _vfe(x2d, nf, W, wbig, sel, selt, e, bd,
                b.reshape(1, C_OUT), gamma.reshape(1, C_OUT),
                beta.reshape(1, C_OUT))


# speedup vs baseline: 1.1687x; 1.0140x over previous
import jax
import jax.numpy as jnp
from jax import lax
from jax.experimental import pallas as pl
from jax.experimental.pallas import tpu as pltpu

M, P, C_IN, C_OUT = 40000, 32, 4, 64
LANES = P * C_IN
KAUG = LANES + 8
WIDE = P * C_OUT
BM = 4000
NB = M // BM
EPS = 1e-5


def _vfe_kernel(x_ref, n_ref, w_ref, wbig_ref, sel_ref, selt_ref,
                e_ref, bd_ref, b_ref, gamma_ref, beta_ref, out_ref,
                g_acc, s_acc, cnt_acc, xc_ref, w2_ref, rt_ref):
    ph = pl.program_id(0)
    i = pl.program_id(1)

    n = n_ref[:]
    count = jnp.clip(n, 0.0, float(P))

    @pl.when(jnp.logical_and(ph == 0, i == 0))
    def _init():
        g_acc[:] = jnp.zeros_like(g_acc)
        s_acc[:] = jnp.zeros_like(s_acc)
        cnt_acc[0, 0] = 0.0

    @pl.when(ph == 0)
    def _accumulate():
        pidx = (lax.broadcasted_iota(jnp.int32, (BM, LANES), 1) // C_IN
                ).astype(jnp.float32)
        xm = jnp.where(pidx < n, x_ref[:], 0.0)
        g_acc[:] += lax.dot_general(
            xm, xm, (((0,), (0,)), ((), ())),
            preferred_element_type=jnp.float32)
        s_acc[:] += jnp.sum(xm, axis=0, keepdims=True)
        cnt_acc[0, 0] += jnp.sum(count)
        row = pl.ds(i * BM, BM)
        xc_ref[row, :LANES] = xm.astype(jnp.bfloat16)
        ones_col = jnp.where(
            lax.broadcasted_iota(jnp.int32, (BM, 8), 1) == 0, 1.0, 0.0)
        xc_ref[row, LANES:] = ones_col.astype(jnp.bfloat16)

    @pl.when(jnp.logical_and(ph == 1, i == 0))
    def _finalize_stats():
        cnt = cnt_acc[0, 0]
        nv = jnp.maximum(cnt, 1.0)
        gm = g_acc[:] * bd_ref[:]
        sxx = jnp.dot(
            lax.dot_general(e_ref[:], gm, (((0,), (0,)), ((), ())),
                            preferred_element_type=jnp.float32),
            e_ref[:], preferred_element_type=jnp.float32)
        s4 = jnp.dot(s_acc[:], e_ref[:],
                     preferred_element_type=jnp.float32)
        sw = jnp.dot(s4, w_ref[:],
                     preferred_element_type=jnp.float32)
        bvec = b_ref[:]
        mean = (sw + cnt * bvec) / nv
        t4 = jnp.dot(sxx, w_ref[:],
                     preferred_element_type=jnp.float32)
        q = (jnp.sum(w_ref[:] * t4, axis=0, keepdims=True)
             + 2.0 * bvec * sw + cnt * bvec * bvec)
        var = q / nv - mean * mean
        scale = gamma_ref[:] * lax.rsqrt(var + EPS)
        shift = beta_ref[:] - mean * scale
        tb = scale * bvec + shift
        tbh = tb.astype(jnp.bfloat16).astype(jnp.float32)
        rt_ref[:] = jnp.maximum(tbh, 0.0)
        s2048 = jnp.dot(scale, selt_ref[:],
                        preferred_element_type=jnp.float32)
        t2048 = jnp.dot(tb, selt_ref[:],
                        preferred_element_type=jnp.float32)
        w2_ref[:LANES, :] = (wbig_ref[:] * s2048).astype(jnp.bfloat16)
        trow = jnp.where(
            lax.broadcasted_iota(jnp.int32, (8, WIDE), 0) == 0,
            t2048, 0.0)
        w2_ref[LANES:, :] = trow.astype(jnp.bfloat16)

    @pl.when(ph == 1)
    def _emit():
        xc = xc_ref[pl.ds(i * BM, BM), :]
        act = jnp.maximum(
            jnp.dot(xc, w2_ref[:], preferred_element_type=jnp.float32),
            0.0)
        summed = jnp.dot(act.astype(jnp.bfloat16), sel_ref[:],
                         preferred_element_type=jnp.float32)
        summed = summed - (float(P) - count) * rt_ref[:]
        inv = jnp.where(count > 0.0, 1.0 / jnp.maximum(count, 1.0), 0.0)
        out_ref[:] = summed * inv


@jax.jit
def _vfe(x2d, nf, w, wbig, sel, selt, e, bd, b2, gamma2, beta2):
    return pl.pallas_call(
        _vfe_kernel,
        grid=(2, NB),
        in_specs=[
            pl.BlockSpec((BM, LANES), lambda ph, i: ((1 - ph) * i, 0)),
            pl.BlockSpec((BM, 1), lambda ph, i: (i, 0)),
            pl.BlockSpec((C_IN, C_OUT), lambda ph, i: (0, 0)),
            pl.BlockSpec((LANES, WIDE), lambda ph, i: (0, 0)),
            pl.BlockSpec((WIDE, C_OUT), lambda ph, i: (0, 0)),
            pl.BlockSpec((C_OUT, WIDE), lambda ph, i: (0, 0)),
            pl.BlockSpec((LANES, C_IN), lambda ph, i: (0, 0)),
            pl.BlockSpec((LANES, LANES), lambda ph, i: (0, 0)),
            pl.BlockSpec((1, C_OUT), lambda ph, i: (0, 0)),
            pl.BlockSpec((1, C_OUT), lambda ph, i: (0, 0)),
            pl.BlockSpec((1, C_OUT), lambda ph, i: (0, 0)),
        ],
        out_specs=pl.BlockSpec((BM, C_OUT), lambda ph, i: (ph * i, 0)),
        out_shape=jax.ShapeDtypeStruct((M, C_OUT), jnp.float32),
        scratch_shapes=[
            pltpu.VMEM((LANES, LANES), jnp.float32),
            pltpu.VMEM((1, LANES), jnp.float32),
            pltpu.SMEM((1, 1), jnp.float32),
            pltpu.VMEM((M, KAUG), jnp.bfloat16),
            pltpu.VMEM((KAUG, WIDE), jnp.bfloat16),
            pltpu.VMEM((1, C_OUT), jnp.float32),
        ],
    )(x2d, nf, w, wbig, sel, selt, e, bd, b2, gamma2, beta2)


def kernel(voxel_features, voxel_num_points, W, b, gamma, beta):
    x2d = voxel_features.reshape(M, LANES)
    nf = jnp.asarray(voxel_num_points).astype(jnp.float32).reshape(M, 1)
    eye_p = jnp.eye(P, dtype=jnp.float32)
    wbig = jnp.kron(eye_p, W)
    self32 = jnp.kron(jnp.ones((P, 1), jnp.float32),
                      jnp.eye(C_OUT, dtype=jnp.float32))
    sel = self32.astype(jnp.bfloat16)
    selt = self32.T
    e = jnp.tile(jnp.eye(C_IN, dtype=jnp.float32), (P, 1))
    bd = jnp.kron(eye_p, jnp.ones((C_IN, C_IN), jnp.float32))
    return _vfe(x2d, nf, W, wbig, sel, selt, e, bd,
                b.reshape(1, C_OUT), gamma.reshape(1, C_OUT),
                beta.reshape(1, C_OUT))
